# baseline (speedup 1.0000x reference)
import jax
import jax.numpy as jnp
from jax.experimental import pallas as pl
from jax.experimental.pallas import tpu as pltpu


def _phi_body(v_ref, s_ref, t_ref, o_hbm, os_ref, vbuf, sem):
    i = pl.program_id(0)
    nsteps = pl.num_programs(0)
    slot = jax.lax.rem(i, 2)

    th = t_ref[0, 0]
    c = 0.25 * th
    ts_col = (0.5 * th) * jnp.transpose(s_ref[...])
    os_ref[...] = jnp.full((1, 1), s_ref.shape[1], jnp.int32)

    @pl.when(i >= 2)
    def _wait_prev():
        pltpu.make_async_copy(vbuf.at[slot], o_hbm.at[i - 2], sem.at[slot]).wait()

    v = v_ref[...]
    for t in range(16):
        r = 2 * t
        x = v[r:r + 1, :]
        y = v[r + 1:r + 2, :]
        ta = c * (y - x)
        tb = c * (y + x)
        w = ta - jnp.abs(ts_col - tb)
        vbuf[slot, :, 128 * t:128 * (t + 1)] = 0.5 + 0.5 * jnp.tanh(w)

    pltpu.make_async_copy(vbuf.at[slot], o_hbm.at[i], sem.at[slot]).start()

    @pl.when(i == nsteps - 1)
    def _drain():
        pltpu.make_async_copy(vbuf.at[slot], o_hbm.at[i], sem.at[slot]).wait()

        @pl.when(i >= 1)
        def _drain_other():
            other = 1 - slot
            pltpu.make_async_copy(
                vbuf.at[other], o_hbm.at[i - 1], sem.at[other]).wait()


def kernel(diagrams, samples, theta):
    n, p, _ = diagrams.shape
    s = samples.shape[0]

    v = diagrams.reshape(n, p // 128, 128, 2).transpose(0, 1, 3, 2)
    v = v.reshape(n * (p // 128) * 2, 128)
    s2 = samples.reshape(1, s)
    t2 = jnp.reshape(theta, (1, 1))

    out3, oshape = pl.pallas_call(
        _phi_body,
        grid=(n,),
        in_specs=[
            pl.BlockSpec(((p // 128) * 2, 128), lambda i: (i, 0)),
            pl.BlockSpec((1, s), lambda i: (0, 0)),
            pl.BlockSpec((1, 1), lambda i: (0, 0)),
        ],
        out_specs=[
            pl.BlockSpec(memory_space=pltpu.MemorySpace.HBM),
            pl.BlockSpec((1, 1), lambda i: (0, 0)),
        ],
        out_shape=[
            jax.ShapeDtypeStruct((n, s, p), jnp.float32),
            jax.ShapeDtypeStruct((1, 1), jnp.int32),
        ],
        scratch_shapes=[
            pltpu.VMEM((2, s, p), jnp.float32),
            pltpu.SemaphoreType.DMA((2,)),
        ],
    )(v, s2, t2)

    output = out3.transpose(0, 2, 1)
    output_shape = oshape.reshape(1)
    return (output, output_shape)


# speedup vs baseline: 1.8618x; 1.8618x over previous
import jax
import jax.numpy as jnp
from jax.experimental import pallas as pl
from jax.experimental.pallas import tpu as pltpu


def _phi_body(v_ref, s_ref, t_ref, o_hbm, os_ref, vbuf, sem):
    th = t_ref[0, 0]
    c = 0.25 * th
    ts_col = (0.5 * th) * jnp.transpose(s_ref[...])
    os_ref[...] = jnp.full((1, 1), s_ref.shape[1], jnp.int32)

    v = v_ref[...]
    n = vbuf.shape[0]
    for d in range(n):
        for t in range(16):
            r = 32 * d + 2 * t
            x = v[r:r + 1, :]
            y = v[r + 1:r + 2, :]
            ta = c * (y - x)
            tb = c * (y + x)
            w = ta - jnp.abs(ts_col - tb)
            vbuf[d, :, 128 * t:128 * (t + 1)] = 0.5 + 0.5 * jnp.tanh(w)
        pltpu.make_async_copy(vbuf.at[d], o_hbm.at[d], sem.at[d]).start()
    for d in range(n):
        pltpu.make_async_copy(vbuf.at[d], o_hbm.at[d], sem.at[d]).wait()


def kernel(diagrams, samples, theta):
    n, p, _ = diagrams.shape
    s = samples.shape[0]

    v = diagrams.reshape(n, p // 128, 128, 2).transpose(0, 1, 3, 2)
    v = v.reshape(n * (p // 128) * 2, 128)
    s2 = samples.reshape(1, s)
    t2 = jnp.reshape(theta, (1, 1))

    out3, oshape = pl.pallas_call(
        _phi_body,
        in_specs=[
            pl.BlockSpec(memory_space=pltpu.MemorySpace.VMEM),
            pl.BlockSpec(memory_space=pltpu.MemorySpace.VMEM),
            pl.BlockSpec(memory_space=pltpu.MemorySpace.VMEM),
        ],
        out_specs=[
            pl.BlockSpec(memory_space=pltpu.MemorySpace.HBM),
            pl.BlockSpec(memory_space=pltpu.MemorySpace.VMEM),
        ],
        out_shape=[
            jax.ShapeDtypeStruct((n, s, p), jnp.float32),
            jax.ShapeDtypeStruct((1, 1), jnp.int32),
        ],
        scratch_shapes=[
            pltpu.VMEM((n, s, p), jnp.float32),
            pltpu.SemaphoreType.DMA((n,)),
        ],
    )(v, s2, t2)

    output = out3.transpose(0, 2, 1)
    output_shape = oshape.reshape(1)
    return (output, output_shape)


# speedup vs baseline: 1.8713x; 1.0051x over previous
import jax
import jax.numpy as jnp
from jax.experimental import pallas as pl


_G = 8


def _phi_body(v_ref, s_ref, t_ref, o_ref, os_ref):
    th = t_ref[0, 0]
    c = 0.25 * th
    v = v_ref[...]
    ts_col = (0.5 * th) * jnp.transpose(s_ref[...])
    os_ref[...] = jnp.full((1, 1), s_ref.shape[1], jnp.int32)
    for g in range(_G):
        for t in range(16):
            r = 32 * g + 2 * t
            x = v[r:r + 1, :]
            y = v[r + 1:r + 2, :]
            ta = c * (y - x)
            tb = c * (y + x)
            w = ta - jnp.abs(ts_col - tb)
            o_ref[g, :, 128 * t:128 * (t + 1)] = 0.5 + 0.5 * jnp.tanh(w)


def kernel(diagrams, samples, theta):
    n, p, _ = diagrams.shape
    s = samples.shape[0]

    v = diagrams.reshape(n, p // 128, 128, 2).transpose(0, 1, 3, 2)
    v = v.reshape(n * (p // 128) * 2, 128)
    s2 = samples.reshape(1, s)
    t2 = jnp.reshape(theta, (1, 1))

    out3, oshape = pl.pallas_call(
        _phi_body,
        grid=(n // _G,),
        in_specs=[
            pl.BlockSpec(((p // 128) * 2 * _G, 128), lambda i: (i, 0)),
            pl.BlockSpec((1, s), lambda i: (0, 0)),
            pl.BlockSpec((1, 1), lambda i: (0, 0)),
        ],
        out_specs=[
            pl.BlockSpec((_G, s, p), lambda i: (i, 0, 0)),
            pl.BlockSpec((1, 1), lambda i: (0, 0)),
        ],
        out_shape=[
            jax.ShapeDtypeStruct((n, s, p), jnp.float32),
            jax.ShapeDtypeStruct((1, 1), jnp.int32),
        ],
    )(v, s2, t2)

    output = out3.transpose(0, 2, 1)
    output_shape = oshape.reshape(1)
    return (output, output_shape)
